# Initial kernel scaffold; baseline (speedup 1.0000x reference)
import jax
import jax.numpy as jnp
from jax.experimental import pallas as pl


def kernel(x, edge_index, batch, W1_rel, b1_rel, W1_root, W2_rel, b2_rel, W2_root, W3_rel, b3_rel, W3_root, bn1_g, bn1_b, bn2_g, bn2_b, Wc1, bc1, Wc2, bc2):
    raise NotImplementedError("write your pallas kernel here")


# speedup vs baseline: 5.4278x; 5.4278x over previous
import functools

import jax
import jax.numpy as jnp
from jax import lax
from jax.experimental import pallas as pl
from jax.experimental.pallas import tpu as pltpu
from jax.experimental.pallas import tpu_sc as plsc

N = 10000
E = 320000
D_IN = 128
H = 256
G = 128
OUT = 2
HH = H // 2

NSC = 2
NTILE = 16
ROWS_PER_TILE = 640
N_TAIL = N - (NTILE - 1) * ROWS_PER_TILE
N_PAD = NTILE * ROWS_PER_TILE
EDGE_CHUNK = 200

BR = 400
NBLK = N // BR

_P = lax.Precision.HIGHEST


def _dot(a, b):
    return jnp.dot(a, b, preferred_element_type=jnp.float32, precision=_P)


def _edge_loop(src_hbm, dst_hbm, gather_from, acc, srcv, dstv, rows, sem,
               e_base, n_chunks):
    def body(k, carry):
        off = e_base + k * EDGE_CHUNK
        pltpu.sync_copy(src_hbm.at[pl.ds(off, EDGE_CHUNK)], srcv)
        pltpu.sync_copy(dst_hbm.at[pl.ds(off, EDGE_CHUNK)], dstv)
        pltpu.async_copy(gather_from.at[srcv], rows, sem).wait()
        pltpu.sync_copy(rows, acc.at[dstv], add=True)
        return carry

    lax.fori_loop(0, n_chunks, body, 0, unroll=False)


def _spmm_init_acc(zeros_hbm, acc, s):
    pltpu.sync_copy(zeros_hbm, acc.at[pl.ds(s * ROWS_PER_TILE, ROWS_PER_TILE)])
    plsc.subcore_barrier()


def _spmm_readout(acc, out_hbm, c, s):
    plsc.subcore_barrier()

    @pl.when(s < NTILE - 1)
    def _():
        sl = pl.ds(s * ROWS_PER_TILE, ROWS_PER_TILE)
        pltpu.sync_copy(acc.at[sl], out_hbm.at[c, sl])

    @pl.when(s == NTILE - 1)
    def _():
        sl = pl.ds((NTILE - 1) * ROWS_PER_TILE, N_TAIL)
        pltpu.sync_copy(acc.at[sl], out_hbm.at[c, sl])


@functools.cache
def _make_spmm_edge_split():
    mesh = plsc.VectorSubcoreMesh(core_axis_name="c", subcore_axis_name="s")

    @functools.partial(
        pl.kernel,
        out_type=jax.ShapeDtypeStruct((NSC, N, D_IN), jnp.float32),
        mesh=mesh,
        scratch_types=[
            pltpu.VMEM_SHARED((N_PAD, D_IN), jnp.float32),
            pltpu.VMEM((EDGE_CHUNK,), jnp.int32),
            pltpu.VMEM((EDGE_CHUNK,), jnp.int32),
            pltpu.VMEM((EDGE_CHUNK, D_IN), jnp.float32),
            pltpu.SemaphoreType.DMA,
        ],
    )
    def spmm_edge_split(src_hbm, dst_hbm, x_hbm, zeros_hbm, out_hbm,
                        acc, srcv, dstv, rows, sem):
        c = lax.axis_index("c")
        s = lax.axis_index("s")
        _spmm_init_acc(zeros_hbm, acc, s)
        edges_per_tile = E // (NSC * NTILE)
        e_base = (c * NTILE + s) * edges_per_tile
        _edge_loop(src_hbm, dst_hbm, x_hbm, acc, srcv, dstv, rows, sem,
                   e_base, edges_per_tile // EDGE_CHUNK)
        _spmm_readout(acc, out_hbm, c, s)

    return spmm_edge_split


def _spmm_edge_split(src, dst, x, zeros):
    return _make_spmm_edge_split()(src, dst, x, zeros)


@functools.cache
def _make_spmm_feat_split():
    mesh = plsc.VectorSubcoreMesh(core_axis_name="c", subcore_axis_name="s")

    @functools.partial(
        pl.kernel,
        out_type=jax.ShapeDtypeStruct((NSC, N, HH), jnp.float32),
        mesh=mesh,
        scratch_types=[
            pltpu.VMEM_SHARED((N_PAD, HH), jnp.float32),
            pltpu.VMEM((EDGE_CHUNK,), jnp.int32),
            pltpu.VMEM((EDGE_CHUNK,), jnp.int32),
            pltpu.VMEM((EDGE_CHUNK, HH), jnp.float32),
            pltpu.SemaphoreType.DMA,
        ],
    )
    def spmm_feat_split(src_hbm, dst_hbm, hlo_hbm, hhi_hbm, zeros_hbm,
                        out_hbm, acc, srcv, dstv, rows, sem):
        c = lax.axis_index("c")
        s = lax.axis_index("s")
        _spmm_init_acc(zeros_hbm, acc, s)
        edges_per_tile = E // NTILE
        e_base = s * edges_per_tile
        n_chunks = edges_per_tile // EDGE_CHUNK

        @pl.when(c == 0)
        def _():
            _edge_loop(src_hbm, dst_hbm, hlo_hbm, acc, srcv, dstv, rows, sem,
                       e_base, n_chunks)

        @pl.when(c == 1)
        def _():
            _edge_loop(src_hbm, dst_hbm, hhi_hbm, acc, srcv, dstv, rows, sem,
                       e_base, n_chunks)

        _spmm_readout(acc, out_hbm, c, s)

    return spmm_feat_split


def _spmm_feat_split(src, dst, hlo, hhi, zeros):
    return _make_spmm_feat_split()(src, dst, hlo, hhi, zeros)


def _acc_out(ref, val):
    @pl.when(pl.program_id(0) == 0)
    def _():
        ref[...] = val

    @pl.when(pl.program_id(0) > 0)
    def _():
        ref[...] += val


def _stats(z, stats_ref):
    s1 = jnp.sum(z, axis=0, keepdims=True)
    s2 = jnp.sum(z * z, axis=0, keepdims=True)
    _acc_out(stats_ref, jnp.concatenate([s1, s2], axis=0))


def _dense1_body(agg0, agg1, x, wr, b, wo, z_ref, stats_ref):
    a = agg0[0] + agg1[0]
    z = _dot(a, wr[...]) + _dot(x[...], wo[...]) + b[...]
    z_ref[...] = z
    _stats(z, stats_ref)


def _dense23_body(agglo, agghi, hlo, hhi, wr, b, wo, z_ref, stats_ref=None,
                  *, last):
    z = (_dot(agglo[0], wr[:HH, :]) + _dot(agghi[0], wr[HH:, :])
         + _dot(hlo[...], wo[:HH, :]) + _dot(hhi[...], wo[HH:, :]) + b[...])
    if last:
        z_ref[...] = jnp.maximum(z, 0.0)
    else:
        z_ref[...] = z
        _stats(z, stats_ref)


def _norm_body(z, stats, gam, bet, hlo_ref, hhi_ref):
    n = jnp.float32(N)
    mu = stats[0:1, :] / n
    var = stats[1:2, :] / n - mu * mu
    scale = gam[...] * lax.rsqrt(var + 1e-5)
    h = jnp.maximum((z[...] - mu) * scale + bet[...], 0.0)
    hlo_ref[...] = h[:, :HH]
    hhi_ref[...] = h[:, HH:]


def _pool_body(h3, bat, sums_ref, cnts_ref):
    ids = lax.broadcasted_iota(jnp.int32, (1, G), 1)
    onehot = (bat[...] == ids).astype(jnp.float32)
    dn = (((0,), (0,)), ((), ()))
    ps = lax.dot_general(onehot, h3[...], dn,
                         preferred_element_type=jnp.float32, precision=_P)
    pc = lax.dot_general(onehot, jnp.ones((BR, 1), jnp.float32), dn,
                         preferred_element_type=jnp.float32, precision=_P)
    _acc_out(sums_ref, ps)
    _acc_out(cnts_ref, pc)


def _cls_body(sums, cnts, wc1, bc1, wc2, bc2, out_ref):
    p = sums[...] / jnp.maximum(cnts[...], 1.0)
    t = jnp.maximum(_dot(p, wc1[...]) + bc1[...], 0.0)
    out_ref[...] = _dot(t, wc2[...]) + bc2[...]


def _row_blocked(width):
    return pl.BlockSpec((BR, width), lambda i: (i, 0))


def _full(shape):
    return pl.BlockSpec(shape, lambda i: tuple(0 for _ in shape))


def _plane(p, width):
    return pl.BlockSpec((1, BR, width), lambda i, _p=p: (_p, i, 0))


def _dense1(agg, x, wr, b, wo):
    return pl.pallas_call(
        _dense1_body,
        grid=(NBLK,),
        in_specs=[_plane(0, D_IN), _plane(1, D_IN), _row_blocked(D_IN),
                  _full((D_IN, H)), _full((1, H)), _full((D_IN, H))],
        out_specs=[_row_blocked(H), _full((2, H))],
        out_shape=[jax.ShapeDtypeStruct((N, H), jnp.float32),
                   jax.ShapeDtypeStruct((2, H), jnp.float32)],
    )(agg, agg, x, wr, b.reshape(1, H), wo)


def _dense23(agg, hlo, hhi, wr, b, wo, last):
    out_specs = [_row_blocked(H)]
    out_shape = [jax.ShapeDtypeStruct((N, H), jnp.float32)]
    if not last:
        out_specs.append(_full((2, H)))
        out_shape.append(jax.ShapeDtypeStruct((2, H), jnp.float32))
    res = pl.pallas_call(
        functools.partial(_dense23_body, last=last),
        grid=(NBLK,),
        in_specs=[_plane(0, HH), _plane(1, HH), _row_blocked(HH),
                  _row_blocked(HH), _full((H, H)), _full((1, H)),
                  _full((H, H))],
        out_specs=out_specs,
        out_shape=out_shape,
    )(agg, agg, hlo, hhi, wr, b.reshape(1, H), wo)
    return res if not last else (res[0], None)


def _norm(z, stats, gam, bet):
    return pl.pallas_call(
        _norm_body,
        grid=(NBLK,),
        in_specs=[_row_blocked(H), _full((2, H)), _full((1, H)),
                  _full((1, H))],
        out_specs=[_row_blocked(HH), _row_blocked(HH)],
        out_shape=[jax.ShapeDtypeStruct((N, HH), jnp.float32),
                   jax.ShapeDtypeStruct((N, HH), jnp.float32)],
    )(z, stats, gam.reshape(1, H), bet.reshape(1, H))


def _pool(h3, batch2d):
    return pl.pallas_call(
        _pool_body,
        grid=(NBLK,),
        in_specs=[_row_blocked(H), _row_blocked(1)],
        out_specs=[_full((G, H)), _full((G, 1))],
        out_shape=[jax.ShapeDtypeStruct((G, H), jnp.float32),
                   jax.ShapeDtypeStruct((G, 1), jnp.float32)],
    )(h3, batch2d)


def _cls(sums, cnts, wc1, bc1, wc2, bc2):
    return pl.pallas_call(
        _cls_body,
        in_specs=[_full((G, H)), _full((G, 1)), _full((H, H)), _full((1, H)),
                  _full((H, OUT)), _full((1, OUT))],
        out_specs=_full((G, OUT)),
        out_shape=jax.ShapeDtypeStruct((G, OUT), jnp.float32),
        grid=(1,),
    )(sums, cnts, wc1, bc1.reshape(1, H), wc2, bc2.reshape(1, OUT))


def kernel(x, edge_index, batch,
           W1_rel, b1_rel, W1_root,
           W2_rel, b2_rel, W2_root,
           W3_rel, b3_rel, W3_root,
           bn1_g, bn1_b, bn2_g, bn2_b,
           Wc1, bc1, Wc2, bc2):
    src = edge_index[0]
    dst = edge_index[1]
    zeros = jnp.zeros((ROWS_PER_TILE, D_IN), jnp.float32)
    batch2d = batch.reshape(N, 1)

    agg1 = _spmm_edge_split(src, dst, x, zeros)
    z1, st1 = _dense1(agg1, x, W1_rel, b1_rel, W1_root)
    h1lo, h1hi = _norm(z1, st1, bn1_g, bn1_b)

    agg2 = _spmm_feat_split(src, dst, h1lo, h1hi, zeros)
    z2, st2 = _dense23(agg2, h1lo, h1hi, W2_rel, b2_rel, W2_root, last=False)
    h2lo, h2hi = _norm(z2, st2, bn2_g, bn2_b)

    agg3 = _spmm_feat_split(src, dst, h2lo, h2hi, zeros)
    h3, _ = _dense23(agg3, h2lo, h2hi, W3_rel, b3_rel, W3_root, last=True)

    sums, cnts = _pool(h3, batch2d)
    return _cls(sums, cnts, Wc1, bc1, Wc2, bc2)


# speedup vs baseline: 6.0949x; 1.1229x over previous
import functools

import jax
import jax.numpy as jnp
from jax import lax
from jax.experimental import pallas as pl
from jax.experimental.pallas import tpu as pltpu
from jax.experimental.pallas import tpu_sc as plsc

N = 10000
E = 320000
D_IN = 128
H = 256
G = 128
OUT = 2
HH = H // 2

NSC = 2
NTILE = 16
ROWS_PER_TILE = 640
N_TAIL = N - (NTILE - 1) * ROWS_PER_TILE
N_PAD = NTILE * ROWS_PER_TILE
EDGE_CHUNK = 40
NBUF = 5

BR = 400
NBLK = N // BR

_P = lax.Precision.HIGHEST


def _dot(a, b):
    return jnp.dot(a, b, preferred_element_type=jnp.float32, precision=_P)


def _edge_loop(pairs_hbm, gather_from, acc, ibufs, rows,
               isems, gsems, ssems, chunk_base, n_chunks):
    n_groups = n_chunks // NBUF

    def group(g, carry):
        base = chunk_base + g * NBUF
        idescs = [
            pltpu.async_copy(pairs_hbm.at[base + b], ibufs[b], isems[b])
            for b in range(NBUF)
        ]
        gdescs = []
        for b in range(NBUF):
            idescs[b].wait()
            gdescs.append(
                pltpu.async_copy(gather_from.at[ibufs[b].at[0]], rows[b],
                                 gsems[b]))
        sdescs = []
        for b in range(NBUF):
            gdescs[b].wait()
            sdescs.append(
                pltpu.async_copy(rows[b], acc.at[ibufs[b].at[1]], ssems[b],
                                 add=True))
        for b in range(NBUF):
            sdescs[b].wait()
        return carry

    lax.fori_loop(0, n_groups, group, 0, unroll=False)


def _spmm_init_acc(zeros_hbm, acc, s):
    pltpu.sync_copy(zeros_hbm, acc.at[pl.ds(s * ROWS_PER_TILE, ROWS_PER_TILE)])
    plsc.subcore_barrier()


def _spmm_readout(acc, out_hbm, c, s):
    plsc.subcore_barrier()

    @pl.when(s < NTILE - 1)
    def _():
        sl = pl.ds(s * ROWS_PER_TILE, ROWS_PER_TILE)
        pltpu.sync_copy(acc.at[sl], out_hbm.at[c, sl])

    @pl.when(s == NTILE - 1)
    def _():
        sl = pl.ds((NTILE - 1) * ROWS_PER_TILE, N_TAIL)
        pltpu.sync_copy(acc.at[sl], out_hbm.at[c, sl])


@functools.cache
def _make_spmm_edge_split():
    mesh = plsc.VectorSubcoreMesh(core_axis_name="c", subcore_axis_name="s")

    @functools.partial(
        pl.kernel,
        out_type=jax.ShapeDtypeStruct((NSC, N, D_IN), jnp.float32),
        mesh=mesh,
        scratch_types=[
            pltpu.VMEM_SHARED((N_PAD, D_IN), jnp.float32),
            [pltpu.VMEM((2, EDGE_CHUNK), jnp.int32)] * NBUF,
            [pltpu.VMEM((EDGE_CHUNK, D_IN), jnp.float32)] * NBUF,
            [pltpu.SemaphoreType.DMA] * NBUF,
            [pltpu.SemaphoreType.DMA] * NBUF,
            [pltpu.SemaphoreType.DMA] * NBUF,
        ],
    )
    def spmm_edge_split(pairs_hbm, x_hbm, zeros_hbm, out_hbm,
                        acc, ibufs, rows, isems, gsems, ssems):
        c = lax.axis_index("c")
        s = lax.axis_index("s")
        _spmm_init_acc(zeros_hbm, acc, s)
        chunks_per_tile = E // (NSC * NTILE * EDGE_CHUNK)
        chunk_base = (c * NTILE + s) * chunks_per_tile
        _edge_loop(pairs_hbm, x_hbm, acc, ibufs, rows,
                   isems, gsems, ssems, chunk_base, chunks_per_tile)
        _spmm_readout(acc, out_hbm, c, s)

    return spmm_edge_split


def _spmm_edge_split(pairs, x, zeros):
    return _make_spmm_edge_split()(pairs, x, zeros)


@functools.cache
def _make_spmm_feat_split():
    mesh = plsc.VectorSubcoreMesh(core_axis_name="c", subcore_axis_name="s")

    @functools.partial(
        pl.kernel,
        out_type=jax.ShapeDtypeStruct((NSC, N, HH), jnp.float32),
        mesh=mesh,
        scratch_types=[
            pltpu.VMEM_SHARED((N_PAD, HH), jnp.float32),
            [pltpu.VMEM((2, EDGE_CHUNK), jnp.int32)] * NBUF,
            [pltpu.VMEM((EDGE_CHUNK, HH), jnp.float32)] * NBUF,
            [pltpu.SemaphoreType.DMA] * NBUF,
            [pltpu.SemaphoreType.DMA] * NBUF,
            [pltpu.SemaphoreType.DMA] * NBUF,
        ],
    )
    def spmm_feat_split(pairs_hbm, hlo_hbm, hhi_hbm, zeros_hbm,
                        out_hbm, acc, ibufs, rows, isems, gsems, ssems):
        c = lax.axis_index("c")
        s = lax.axis_index("s")
        _spmm_init_acc(zeros_hbm, acc, s)
        chunks_per_tile = E // (NTILE * EDGE_CHUNK)
        chunk_base = s * chunks_per_tile

        @pl.when(c == 0)
        def _():
            _edge_loop(pairs_hbm, hlo_hbm, acc, ibufs, rows,
                       isems, gsems, ssems, chunk_base, chunks_per_tile)

        @pl.when(c == 1)
        def _():
            _edge_loop(pairs_hbm, hhi_hbm, acc, ibufs, rows,
                       isems, gsems, ssems, chunk_base, chunks_per_tile)

        _spmm_readout(acc, out_hbm, c, s)

    return spmm_feat_split


def _spmm_feat_split(pairs, hlo, hhi, zeros):
    return _make_spmm_feat_split()(pairs, hlo, hhi, zeros)


def _acc_out(ref, val):
    @pl.when(pl.program_id(0) == 0)
    def _():
        ref[...] = val

    @pl.when(pl.program_id(0) > 0)
    def _():
        ref[...] += val


def _stats(z, stats_ref):
    s1 = jnp.sum(z, axis=0, keepdims=True)
    s2 = jnp.sum(z * z, axis=0, keepdims=True)
    _acc_out(stats_ref, jnp.concatenate([s1, s2], axis=0))


def _dense1_body(agg0, agg1, x, wr, b, wo, z_ref, stats_ref):
    a = agg0[0] + agg1[0]
    z = _dot(a, wr[...]) + _dot(x[...], wo[...]) + b[...]
    z_ref[...] = z
    _stats(z, stats_ref)


def _dense23_body(agglo, agghi, hlo, hhi, wr, b, wo, z_ref, stats_ref=None,
                  *, last):
    z = (_dot(agglo[0], wr[:HH, :]) + _dot(agghi[0], wr[HH:, :])
         + _dot(hlo[...], wo[:HH, :]) + _dot(hhi[...], wo[HH:, :]) + b[...])
    if last:
        z_ref[...] = jnp.maximum(z, 0.0)
    else:
        z_ref[...] = z
        _stats(z, stats_ref)


def _norm_body(z, stats, gam, bet, hlo_ref, hhi_ref):
    n = jnp.float32(N)
    mu = stats[0:1, :] / n
    var = stats[1:2, :] / n - mu * mu
    scale = gam[...] * lax.rsqrt(var + 1e-5)
    h = jnp.maximum((z[...] - mu) * scale + bet[...], 0.0)
    hlo_ref[...] = h[:, :HH]
    hhi_ref[...] = h[:, HH:]


def _pool_body(h3, bat, sums_ref, cnts_ref):
    ids = lax.broadcasted_iota(jnp.int32, (1, G), 1)
    onehot = (bat[...] == ids).astype(jnp.float32)
    dn = (((0,), (0,)), ((), ()))
    ps = lax.dot_general(onehot, h3[...], dn,
                         preferred_element_type=jnp.float32, precision=_P)
    pc = lax.dot_general(onehot, jnp.ones((BR, 1), jnp.float32), dn,
                         preferred_element_type=jnp.float32, precision=_P)
    _acc_out(sums_ref, ps)
    _acc_out(cnts_ref, pc)


def _cls_body(sums, cnts, wc1, bc1, wc2, bc2, out_ref):
    p = sums[...] / jnp.maximum(cnts[...], 1.0)
    t = jnp.maximum(_dot(p, wc1[...]) + bc1[...], 0.0)
    out_ref[...] = _dot(t, wc2[...]) + bc2[...]


def _row_blocked(width):
    return pl.BlockSpec((BR, width), lambda i: (i, 0))


def _full(shape):
    return pl.BlockSpec(shape, lambda i: tuple(0 for _ in shape))


def _plane(p, width):
    return pl.BlockSpec((1, BR, width), lambda i, _p=p: (_p, i, 0))


def _dense1(agg, x, wr, b, wo):
    return pl.pallas_call(
        _dense1_body,
        grid=(NBLK,),
        in_specs=[_plane(0, D_IN), _plane(1, D_IN), _row_blocked(D_IN),
                  _full((D_IN, H)), _full((1, H)), _full((D_IN, H))],
        out_specs=[_row_blocked(H), _full((2, H))],
        out_shape=[jax.ShapeDtypeStruct((N, H), jnp.float32),
                   jax.ShapeDtypeStruct((2, H), jnp.float32)],
    )(agg, agg, x, wr, b.reshape(1, H), wo)


def _dense23(agg, hlo, hhi, wr, b, wo, last):
    out_specs = [_row_blocked(H)]
    out_shape = [jax.ShapeDtypeStruct((N, H), jnp.float32)]
    if not last:
        out_specs.append(_full((2, H)))
        out_shape.append(jax.ShapeDtypeStruct((2, H), jnp.float32))
    res = pl.pallas_call(
        functools.partial(_dense23_body, last=last),
        grid=(NBLK,),
        in_specs=[_plane(0, HH), _plane(1, HH), _row_blocked(HH),
                  _row_blocked(HH), _full((H, H)), _full((1, H)),
                  _full((H, H))],
        out_specs=out_specs,
        out_shape=out_shape,
    )(agg, agg, hlo, hhi, wr, b.reshape(1, H), wo)
    return res if not last else (res[0], None)


def _norm(z, stats, gam, bet):
    return pl.pallas_call(
        _norm_body,
        grid=(NBLK,),
        in_specs=[_row_blocked(H), _full((2, H)), _full((1, H)),
                  _full((1, H))],
        out_specs=[_row_blocked(HH), _row_blocked(HH)],
        out_shape=[jax.ShapeDtypeStruct((N, HH), jnp.float32),
                   jax.ShapeDtypeStruct((N, HH), jnp.float32)],
    )(z, stats, gam.reshape(1, H), bet.reshape(1, H))


def _pool(h3, batch2d):
    return pl.pallas_call(
        _pool_body,
        grid=(NBLK,),
        in_specs=[_row_blocked(H), _row_blocked(1)],
        out_specs=[_full((G, H)), _full((G, 1))],
        out_shape=[jax.ShapeDtypeStruct((G, H), jnp.float32),
                   jax.ShapeDtypeStruct((G, 1), jnp.float32)],
    )(h3, batch2d)


def _cls(sums, cnts, wc1, bc1, wc2, bc2):
    return pl.pallas_call(
        _cls_body,
        in_specs=[_full((G, H)), _full((G, 1)), _full((H, H)), _full((1, H)),
                  _full((H, OUT)), _full((1, OUT))],
        out_specs=_full((G, OUT)),
        out_shape=jax.ShapeDtypeStruct((G, OUT), jnp.float32),
        grid=(1,),
    )(sums, cnts, wc1, bc1.reshape(1, H), wc2, bc2.reshape(1, OUT))


def kernel(x, edge_index, batch,
           W1_rel, b1_rel, W1_root,
           W2_rel, b2_rel, W2_root,
           W3_rel, b3_rel, W3_root,
           bn1_g, bn1_b, bn2_g, bn2_b,
           Wc1, bc1, Wc2, bc2):
    pairs = jnp.stack([edge_index[0].reshape(-1, EDGE_CHUNK),
                       edge_index[1].reshape(-1, EDGE_CHUNK)], axis=1)
    zeros = jnp.zeros((ROWS_PER_TILE, D_IN), jnp.float32)
    batch2d = batch.reshape(N, 1)

    agg1 = _spmm_edge_split(pairs, x, zeros)
    z1, st1 = _dense1(agg1, x, W1_rel, b1_rel, W1_root)
    h1lo, h1hi = _norm(z1, st1, bn1_g, bn1_b)

    agg2 = _spmm_feat_split(pairs, h1lo, h1hi, zeros)
    z2, st2 = _dense23(agg2, h1lo, h1hi, W2_rel, b2_rel, W2_root, last=False)
    h2lo, h2hi = _norm(z2, st2, bn2_g, bn2_b)

    agg3 = _spmm_feat_split(pairs, h2lo, h2hi, zeros)
    h3, _ = _dense23(agg3, h2lo, h2hi, W3_rel, b3_rel, W3_root, last=True)

    sums, cnts = _pool(h3, batch2d)
    return _cls(sums, cnts, Wc1, bc1, Wc2, bc2)
